# Initial kernel scaffold; baseline (speedup 1.0000x reference)
import jax
import jax.numpy as jnp
from jax.experimental import pallas as pl


def kernel(x, lengths, T_logits, f_logits):
    raise NotImplementedError("write your pallas kernel here")


# speedup vs baseline: 4.5945x; 4.5945x over previous
import jax
import jax.numpy as jnp
from jax.experimental import pallas as pl
from jax.experimental.pallas import tpu as pltpu

Q = 128
V = 64
B = 16
L = 512


def _fwd_body(x_smem, len_smem, T_ref, f_ref, out_ref, P_ref):
    def softmax_body(v, carry):
        blk = T_ref[v]
        m = jnp.max(blk, axis=-1, keepdims=True)
        e = jnp.exp(blk - m)
        s = jnp.sum(e, axis=-1, keepdims=True)
        P_ref[v] = e / s
        return carry

    jax.lax.fori_loop(0, V, softmax_body, 0)

    init = jnp.where(
        jax.lax.broadcasted_iota(jnp.int32, (1, Q), 1) == 0, 1.0, 0.0
    ).astype(jnp.float32)
    alphas0 = tuple(init for _ in range(B))

    def step(t, alphas):
        new = []
        for b in range(B):
            sym = x_smem[b, t]
            mat = P_ref[sym]
            nb = jax.lax.dot_general(
                alphas[b], mat,
                (((1,), (0,)), ((), ())),
                precision=jax.lax.Precision.HIGHEST,
                preferred_element_type=jnp.float32,
            )
            keep = t < len_smem[b]
            new.append(jnp.where(keep, nb, alphas[b]))
        return tuple(new)

    alphas = jax.lax.fori_loop(0, L, step, alphas0)
    A = jnp.concatenate(alphas, axis=0)

    f = f_ref[...]
    mf = jnp.max(f)
    w = jnp.exp(f - mf)
    s = jnp.sum(A * w, axis=-1, keepdims=True)
    out_ref[...] = jnp.log(s) + mf


def kernel(x, lengths, T_logits, f_logits):
    T_s = jnp.transpose(T_logits, (1, 0, 2))
    f2 = f_logits.reshape(1, Q)
    out = pl.pallas_call(
        _fwd_body,
        out_shape=jax.ShapeDtypeStruct((B, 1), jnp.float32),
        in_specs=[
            pl.BlockSpec(memory_space=pltpu.SMEM),
            pl.BlockSpec(memory_space=pltpu.SMEM),
            pl.BlockSpec(memory_space=pltpu.VMEM),
            pl.BlockSpec(memory_space=pltpu.VMEM),
        ],
        out_specs=pl.BlockSpec(memory_space=pltpu.VMEM),
        scratch_shapes=[pltpu.VMEM((V, Q, Q), jnp.float32)],
    )(x, lengths, T_s, f2)
    return out.reshape(B)


# speedup vs baseline: 7.8165x; 1.7013x over previous
import jax
import jax.numpy as jnp
from jax.experimental import pallas as pl
from jax.experimental.pallas import tpu as pltpu

Q = 128
V = 64
VP = V + 1
B = 16
L = 512


def _fwd_body(x_smem, T_ref, f_ref, out_ref, Phi_ref, Plo_ref):
    def softmax_body(v, carry):
        blk = T_ref[v]
        m = jnp.max(blk, axis=-1, keepdims=True)
        e = jnp.exp(blk - m)
        s = jnp.sum(e, axis=-1, keepdims=True)
        p = e / s
        hi = p.astype(jnp.bfloat16)
        Phi_ref[v] = hi
        Plo_ref[v] = (p - hi.astype(jnp.float32)).astype(jnp.bfloat16)
        return carry

    jax.lax.fori_loop(0, V, softmax_body, 0)

    row = jax.lax.broadcasted_iota(jnp.int32, (Q, Q), 0)
    col = jax.lax.broadcasted_iota(jnp.int32, (Q, Q), 1)
    eye = jnp.where(row == col, 1.0, 0.0).astype(jnp.float32)
    Phi_ref[V] = eye.astype(jnp.bfloat16)
    Plo_ref[V] = jnp.zeros((Q, Q), jnp.bfloat16)

    init = jnp.where(
        jax.lax.broadcasted_iota(jnp.int32, (1, Q), 1) == 0, 1.0, 0.0
    ).astype(jnp.float32)
    alphas0 = tuple(init for _ in range(B))

    dims = (((1,), (0,)), ((), ()))

    def step(t, alphas):
        new = []
        for b in range(B):
            sym = x_smem[b, t]
            mh = Phi_ref[sym]
            ml = Plo_ref[sym]
            a = alphas[b]
            a_hi = a.astype(jnp.bfloat16)
            a_lo = (a - a_hi.astype(jnp.float32)).astype(jnp.bfloat16)
            nb = (
                jax.lax.dot_general(a_hi, mh, dims,
                                    preferred_element_type=jnp.float32)
                + jax.lax.dot_general(a_hi, ml, dims,
                                      preferred_element_type=jnp.float32)
                + jax.lax.dot_general(a_lo, mh, dims,
                                      preferred_element_type=jnp.float32)
            )
            new.append(nb)
        return tuple(new)

    alphas = jax.lax.fori_loop(0, L, step, alphas0)
    A = jnp.concatenate(alphas, axis=0)

    f = f_ref[...]
    mf = jnp.max(f)
    w = jnp.exp(f - mf)
    s = jnp.sum(A * w, axis=-1, keepdims=True)
    out_ref[...] = jnp.log(s) + mf


def kernel(x, lengths, T_logits, f_logits):
    T_s = jnp.transpose(T_logits, (1, 0, 2))
    f2 = f_logits.reshape(1, Q)
    pos = jnp.arange(L, dtype=jnp.int32)[None, :]
    x_eff = jnp.where(pos < lengths[:, None], x, V).astype(jnp.int32)
    out = pl.pallas_call(
        _fwd_body,
        out_shape=jax.ShapeDtypeStruct((B, 1), jnp.float32),
        in_specs=[
            pl.BlockSpec(memory_space=pltpu.SMEM),
            pl.BlockSpec(memory_space=pltpu.VMEM),
            pl.BlockSpec(memory_space=pltpu.VMEM),
        ],
        out_specs=pl.BlockSpec(memory_space=pltpu.VMEM),
        scratch_shapes=[
            pltpu.VMEM((VP, Q, Q), jnp.bfloat16),
            pltpu.VMEM((VP, Q, Q), jnp.bfloat16),
        ],
    )(x_eff, T_s, f2)
    return out.reshape(B)


# speedup vs baseline: 11.7889x; 1.5082x over previous
import jax
import jax.numpy as jnp
from jax.experimental import pallas as pl
from jax.experimental.pallas import tpu as pltpu

Q = 128
V = 64
VP = V + 1
B = 16
L = 512


def _fwd_body(x_smem, T_ref, f_ref, out_ref, P3_ref):
    def softmax_body(v, carry):
        blk = T_ref[v]
        m = jnp.max(blk, axis=-1, keepdims=True)
        e = jnp.exp(blk - m)
        s = jnp.sum(e, axis=-1, keepdims=True)
        p = e / s
        hi = p.astype(jnp.bfloat16)
        lo = (p - hi.astype(jnp.float32)).astype(jnp.bfloat16)
        P3_ref[v, 0:Q, :] = hi
        P3_ref[v, Q:2 * Q, :] = lo
        P3_ref[v, 2 * Q:3 * Q, :] = hi
        return carry

    jax.lax.fori_loop(0, V, softmax_body, 0)

    row = jax.lax.broadcasted_iota(jnp.int32, (Q, Q), 0)
    col = jax.lax.broadcasted_iota(jnp.int32, (Q, Q), 1)
    eye = jnp.where(row == col, 1.0, 0.0).astype(jnp.bfloat16)
    P3_ref[V, 0:Q, :] = eye
    P3_ref[V, Q:2 * Q, :] = jnp.zeros((Q, Q), jnp.bfloat16)
    P3_ref[V, 2 * Q:3 * Q, :] = eye

    init = jnp.where(
        jax.lax.broadcasted_iota(jnp.int32, (1, Q), 1) == 0, 1.0, 0.0
    ).astype(jnp.float32)
    alphas0 = tuple(init for _ in range(B))

    dims = (((1,), (0,)), ((), ()))

    def substep(t, alphas):
        new = []
        for b in range(B):
            sym = x_smem[b, t]
            m3 = P3_ref[sym]
            a = alphas[b]
            a_hi = a.astype(jnp.bfloat16)
            a_lo = (a - a_hi.astype(jnp.float32)).astype(jnp.bfloat16)
            a3 = jnp.concatenate([a_hi, a_hi, a_lo], axis=1)
            nb = jax.lax.dot_general(a3, m3, dims,
                                     preferred_element_type=jnp.float32)
            new.append(nb)
        return tuple(new)

    def step(i, alphas):
        return substep(2 * i + 1, substep(2 * i, alphas))

    alphas = jax.lax.fori_loop(0, L // 2, step, alphas0)
    A = jnp.concatenate(alphas, axis=0)

    f = f_ref[...]
    mf = jnp.max(f)
    w = jnp.exp(f - mf)
    s = jnp.sum(A * w, axis=-1, keepdims=True)
    out_ref[...] = jnp.log(s) + mf


def kernel(x, lengths, T_logits, f_logits):
    T_s = jnp.transpose(T_logits, (1, 0, 2))
    f2 = f_logits.reshape(1, Q)
    pos = jnp.arange(L, dtype=jnp.int32)[None, :]
    x_eff = jnp.where(pos < lengths[:, None], x, V).astype(jnp.int32)
    out = pl.pallas_call(
        _fwd_body,
        out_shape=jax.ShapeDtypeStruct((B, 1), jnp.float32),
        in_specs=[
            pl.BlockSpec(memory_space=pltpu.SMEM),
            pl.BlockSpec(memory_space=pltpu.VMEM),
            pl.BlockSpec(memory_space=pltpu.VMEM),
        ],
        out_specs=pl.BlockSpec(memory_space=pltpu.VMEM),
        scratch_shapes=[
            pltpu.VMEM((VP, 3 * Q, Q), jnp.bfloat16),
        ],
    )(x_eff, T_s, f2)
    return out.reshape(B)


# speedup vs baseline: 13.3682x; 1.1340x over previous
import jax
import jax.numpy as jnp
from jax.experimental import pallas as pl
from jax.experimental.pallas import tpu as pltpu

Q = 128
V = 64
VP = V + 1
B = 16
L = 512


def _fwd_body(x_smem, T_ref, f_ref, out_ref, P3_ref):
    def softmax_body(v, carry):
        blk = T_ref[v]
        m = jnp.max(blk, axis=-1, keepdims=True)
        e = jnp.exp(blk - m)
        s = jnp.sum(e, axis=-1, keepdims=True)
        p = e / s
        hi = p.astype(jnp.bfloat16)
        lo = (p - hi.astype(jnp.float32)).astype(jnp.bfloat16)
        P3_ref[v, 0:Q, :] = hi
        P3_ref[v, Q:2 * Q, :] = lo
        P3_ref[v, 2 * Q:3 * Q, :] = hi
        return carry

    jax.lax.fori_loop(0, V, softmax_body, 0)

    row = jax.lax.broadcasted_iota(jnp.int32, (Q, Q), 0)
    col = jax.lax.broadcasted_iota(jnp.int32, (Q, Q), 1)
    eye = jnp.where(row == col, 1.0, 0.0).astype(jnp.bfloat16)
    P3_ref[V, 0:Q, :] = eye
    P3_ref[V, Q:2 * Q, :] = jnp.zeros((Q, Q), jnp.bfloat16)
    P3_ref[V, 2 * Q:3 * Q, :] = eye

    init = jnp.where(
        jax.lax.broadcasted_iota(jnp.int32, (1, Q), 1) == 0, 1.0, 0.0
    ).astype(jnp.float32)
    alphas0 = tuple(init for _ in range(B))

    dims = (((1,), (0,)), ((), ()))

    def substep(t, alphas):
        new = []
        for b in range(B):
            sym = x_smem[b, t]
            m3 = P3_ref[sym]
            a = alphas[b]
            a_hi = a.astype(jnp.bfloat16)
            a_lo = (a - a_hi.astype(jnp.float32)).astype(jnp.bfloat16)
            a3 = jnp.concatenate([a_hi, a_hi, a_lo], axis=1)
            nb = jax.lax.dot_general(a3, m3, dims,
                                     preferred_element_type=jnp.float32)
            new.append(nb)
        return tuple(new)

    UNROLL = 8

    def step(i, alphas):
        for k in range(UNROLL):
            alphas = substep(UNROLL * i + k, alphas)
        return alphas

    alphas = jax.lax.fori_loop(0, L // UNROLL, step, alphas0)
    A = jnp.concatenate(alphas, axis=0)

    f = f_ref[...]
    mf = jnp.max(f)
    w = jnp.exp(f - mf)
    s = jnp.sum(A * w, axis=-1, keepdims=True)
    out_ref[...] = jnp.log(s) + mf


def kernel(x, lengths, T_logits, f_logits):
    T_s = jnp.transpose(T_logits, (1, 0, 2))
    f2 = f_logits.reshape(1, Q)
    pos = jnp.arange(L, dtype=jnp.int32)[None, :]
    x_eff = jnp.where(pos < lengths[:, None], x, V).astype(jnp.int32)
    out = pl.pallas_call(
        _fwd_body,
        out_shape=jax.ShapeDtypeStruct((B, 1), jnp.float32),
        in_specs=[
            pl.BlockSpec(memory_space=pltpu.SMEM),
            pl.BlockSpec(memory_space=pltpu.VMEM),
            pl.BlockSpec(memory_space=pltpu.VMEM),
        ],
        out_specs=pl.BlockSpec(memory_space=pltpu.VMEM),
        scratch_shapes=[
            pltpu.VMEM((VP, 3 * Q, Q), jnp.bfloat16),
        ],
    )(x_eff, T_s, f2)
    return out.reshape(B)


# speedup vs baseline: 13.6875x; 1.0239x over previous
import jax
import jax.numpy as jnp
from jax.experimental import pallas as pl
from jax.experimental.pallas import tpu as pltpu

Q = 128
V = 64
VP = V + 1
B = 16
L = 512


def _fwd_body(x_smem, T_ref, f_ref, out_ref, P3_ref):
    def softmax_body(v, carry):
        blk = T_ref[v]
        m = jnp.max(blk, axis=-1, keepdims=True)
        e = jnp.exp(blk - m)
        s = jnp.sum(e, axis=-1, keepdims=True)
        p = e / s
        hi = p.astype(jnp.bfloat16)
        lo = (p - hi.astype(jnp.float32)).astype(jnp.bfloat16)
        P3_ref[v, 0:Q, :] = hi
        P3_ref[v, Q:2 * Q, :] = lo
        P3_ref[v, 2 * Q:3 * Q, :] = hi
        return carry

    jax.lax.fori_loop(0, V, softmax_body, 0)

    row = jax.lax.broadcasted_iota(jnp.int32, (Q, Q), 0)
    col = jax.lax.broadcasted_iota(jnp.int32, (Q, Q), 1)
    eye = jnp.where(row == col, 1.0, 0.0).astype(jnp.bfloat16)
    P3_ref[V, 0:Q, :] = eye
    P3_ref[V, Q:2 * Q, :] = jnp.zeros((Q, Q), jnp.bfloat16)
    P3_ref[V, 2 * Q:3 * Q, :] = eye

    init = jnp.where(
        jax.lax.broadcasted_iota(jnp.int32, (1, Q), 1) == 0, 1.0, 0.0
    ).astype(jnp.float32)
    alphas0 = tuple(init for _ in range(B))

    dims = (((1,), (0,)), ((), ()))

    def substep(t, alphas):
        new = []
        for b in range(B):
            sym = x_smem[b, t]
            m3 = P3_ref[sym]
            a = alphas[b]
            a_hi = a.astype(jnp.bfloat16)
            a_lo = (a - a_hi.astype(jnp.float32)).astype(jnp.bfloat16)
            a3 = jnp.concatenate([a_hi, a_hi, a_lo], axis=1)
            nb = jax.lax.dot_general(a3, m3, dims,
                                     preferred_element_type=jnp.float32)
            new.append(nb)
        return tuple(new)

    UNROLL = 16

    def step(i, alphas):
        for k in range(UNROLL):
            alphas = substep(UNROLL * i + k, alphas)
        return alphas

    alphas = jax.lax.fori_loop(0, L // UNROLL, step, alphas0)
    A = jnp.concatenate(alphas, axis=0)

    f = f_ref[...]
    mf = jnp.max(f)
    w = jnp.exp(f - mf)
    s = jnp.sum(A * w, axis=-1, keepdims=True)
    out_ref[...] = jnp.log(s) + mf


def kernel(x, lengths, T_logits, f_logits):
    T_s = jnp.transpose(T_logits, (1, 0, 2))
    f2 = f_logits.reshape(1, Q)
    pos = jnp.arange(L, dtype=jnp.int32)[None, :]
    x_eff = jnp.where(pos < lengths[:, None], x, V).astype(jnp.int32)
    out = pl.pallas_call(
        _fwd_body,
        out_shape=jax.ShapeDtypeStruct((B, 1), jnp.float32),
        in_specs=[
            pl.BlockSpec(memory_space=pltpu.SMEM),
            pl.BlockSpec(memory_space=pltpu.VMEM),
            pl.BlockSpec(memory_space=pltpu.VMEM),
        ],
        out_specs=pl.BlockSpec(memory_space=pltpu.VMEM),
        scratch_shapes=[
            pltpu.VMEM((VP, 3 * Q, Q), jnp.bfloat16),
        ],
    )(x_eff, T_s, f2)
    return out.reshape(B)
